# baseline (speedup 1.0000x reference)
import functools

import jax
import jax.numpy as jnp
from jax import lax
from jax.experimental import pallas as pl
from jax.experimental.pallas import tpu as pltpu
from jax.experimental.pallas import tpu_sc as plsc

B, S, D, H, E, TOP_K = 2, 2048, 1024, 2048, 8, 2
N = B * S
A = N * TOP_K
BLK = 128
PAD = A + E * BLK
NB = PAD // BLK
TB = 512
NTB = N // TB

_SC_NC = 2
_SC_NS = 16
NW = _SC_NC * _SC_NS


DW = D // 2


def _pack_round_bf16(b):
    lsb = (b >> 16) & jnp.uint32(1)
    return (b + jnp.uint32(0x7FFF) + lsb) & jnp.uint32(0xFFFF0000)


def _gating_body(x_ref, wg_ref, bg_ref, g_ref, e_ref, xw_ref):
    x = x_ref[...]
    bits = lax.bitcast_convert_type(x, jnp.uint32)
    xw_ref[...] = lax.bitcast_convert_type(
        _pack_round_bf16(bits[:, :DW]) | (_pack_round_bf16(bits[:, DW:]) >> 16),
        jnp.int32)
    logits = lax.dot_general(x, wg_ref[...], (((1,), (1,)), ((), ())),
                             preferred_element_type=jnp.float32)
    logits = logits + bg_ref[...]
    m = jnp.max(logits, axis=1, keepdims=True)
    p = jnp.exp(logits - m)
    probs = p / jnp.sum(p, axis=1, keepdims=True)
    idx = lax.broadcasted_iota(jnp.int32, probs.shape, 1)
    g1 = jnp.max(probs, axis=1)
    a1 = jnp.min(jnp.where(probs == g1[:, None], idx, E), axis=1)
    masked = jnp.where(idx == a1[:, None], -jnp.inf, probs)
    g2 = jnp.max(masked, axis=1)
    a2 = jnp.min(jnp.where(masked == g2[:, None], idx, E), axis=1)
    g_ref[0] = jnp.stack([g1, g2])
    e_ref[0] = jnp.stack([a1, a2])


def _gating(x_flat, Wg, bg):
    g, e, xw = pl.pallas_call(
        _gating_body,
        grid=(NTB,),
        in_specs=[
            pl.BlockSpec((TB, D), lambda i: (i, 0)),
            pl.BlockSpec((E, D), lambda i: (0, 0)),
            pl.BlockSpec((1, E), lambda i: (0, 0)),
        ],
        out_specs=[
            pl.BlockSpec((1, TOP_K, TB), lambda i: (i, 0, 0)),
            pl.BlockSpec((1, TOP_K, TB), lambda i: (i, 0, 0)),
            pl.BlockSpec((TB, DW), lambda i: (i, 0)),
        ],
        out_shape=[
            jax.ShapeDtypeStruct((NTB, TOP_K, TB), jnp.float32),
            jax.ShapeDtypeStruct((NTB, TOP_K, TB), jnp.int32),
            jax.ShapeDtypeStruct((N, DW), jnp.int32),
        ],
    )(x_flat, Wg, bg.reshape(1, E))
    g = jnp.transpose(g, (0, 2, 1)).reshape(N, TOP_K)
    e = jnp.transpose(e, (0, 2, 1)).reshape(N, TOP_K)
    return g, e, xw


def _route(g, e):
    ea = e.reshape(A)
    one_hot = (ea[:, None] == jnp.arange(E, dtype=jnp.int32)[None, :])
    c = jnp.cumsum(one_hot.astype(jnp.int32), axis=0)
    rank = jnp.take_along_axis(c, ea[:, None], axis=1)[:, 0] - 1
    counts = c[-1]
    padded = ((counts + BLK - 1) // BLK) * BLK
    pstarts = jnp.concatenate([jnp.zeros((1,), jnp.int32),
                               jnp.cumsum(padded)[:-1].astype(jnp.int32)])
    pad_pos = pstarts[ea] + rank
    tok_pad = jnp.zeros((PAD,), jnp.int32).at[pad_pos].set(
        (jnp.arange(A, dtype=jnp.int32) // TOP_K))
    w_pad = jnp.zeros((PAD,), jnp.float32).at[pad_pos].set(g.reshape(A))
    pos = pad_pos
    cumb = jnp.cumsum(padded // BLK).astype(jnp.int32)
    blk_exp = jnp.minimum(
        jnp.searchsorted(cumb, jnp.arange(NB, dtype=jnp.int32), side="right"),
        E - 1).astype(jnp.int32)
    return tok_pad, w_pad, pos, blk_exp


_G_RW = PAD // NW
_G_CH = _G_RW // 4
_G_NCH = _G_RW // _G_CH


@functools.lru_cache(maxsize=None)
def _build_sc_gather():
    mesh = plsc.VectorSubcoreMesh(core_axis_name="c", subcore_axis_name="s")

    @functools.partial(
        pl.kernel,
        out_type=jax.ShapeDtypeStruct((PAD, DW), jnp.int32),
        mesh=mesh,
        scratch_types=[
            pltpu.VMEM((2, _G_CH), jnp.int32),
            pltpu.VMEM((_G_CH, DW), jnp.int32),
            pltpu.VMEM((_G_CH, DW), jnp.int32),
            pltpu.SemaphoreType.DMA,
            pltpu.SemaphoreType.DMA,
            pltpu.SemaphoreType.DMA,
            pltpu.SemaphoreType.DMA,
        ],
    )
    def _sc_gather_rows(x_hbm, idx_hbm, out_hbm, idx_v, r0, r1, g0, g1, w0, w1):
        wid = lax.axis_index("s") * _SC_NC + lax.axis_index("c")
        base = wid * _G_RW
        rows = (r0, r1)
        gsems = (g0, g1)
        wsems = (w0, w1)
        gops = [None, None]
        wops = [None, None]

        def start_gather(b):
            ops = []
            for s0 in range(0, _G_CH, 8):
                ops.append(pltpu.async_copy(
                    x_hbm.at[idx_v.at[b, pl.ds(s0, 8)]],
                    rows[b].at[pl.ds(s0, 8)], gsems[b]))
            return ops

        def start_writeback(b, poff):
            ops = []
            for s0 in range(0, _G_CH, 8):
                ops.append(pltpu.async_copy(
                    rows[b].at[pl.ds(s0, 8)],
                    out_hbm.at[pl.ds(poff + s0, 8)], wsems[b]))
            return ops

        for c in range(_G_NCH):
            b = c & 1
            off = base + c * _G_CH
            if c >= 2:
                for op in wops[b]:
                    op.wait()
            pltpu.sync_copy(idx_hbm.at[pl.ds(off, _G_CH)], idx_v.at[b])
            gops[b] = start_gather(b)
            if c >= 1:
                pb = 1 - b
                for op in gops[pb]:
                    op.wait()
                wops[pb] = start_writeback(pb, base + (c - 1) * _G_CH)
        b = (_G_NCH - 1) & 1
        for op in gops[b]:
            op.wait()
        wops[b] = start_writeback(b, base + (_G_NCH - 1) * _G_CH)
        for op in wops[1 - b]:
            op.wait()
        for op in wops[b]:
            op.wait()

    return _sc_gather_rows


def _mlp_body(s_ref, x_ref, w1_ref, b1_ref, w2_ref, b2_ref, w_ref, y_ref):
    u = lax.bitcast_convert_type(x_ref[...], jnp.uint32)
    xa = lax.bitcast_convert_type(u & jnp.uint32(0xFFFF0000),
                                  jnp.float32).astype(jnp.bfloat16)
    xb = lax.bitcast_convert_type(u << 16, jnp.float32).astype(jnp.bfloat16)
    w1 = w1_ref[0]
    h = lax.dot_general(xa, w1[:, :DW], (((1,), (1,)), ((), ())),
                        preferred_element_type=jnp.float32)
    h = h + lax.dot_general(xb, w1[:, DW:], (((1,), (1,)), ((), ())),
                            preferred_element_type=jnp.float32)
    h = jnp.maximum(h + b1_ref[0], 0.0)
    y = lax.dot_general(h.astype(jnp.bfloat16), w2_ref[0],
                        (((1,), (1,)), ((), ())),
                        preferred_element_type=jnp.float32)
    y_ref[...] = (y + b2_ref[0]) * w_ref[0, 0][:, None]


def _grouped_mlp(x_sorted, W1, b1, W2, b2, w_pad, blk_exp):
    grid_spec = pltpu.PrefetchScalarGridSpec(
        num_scalar_prefetch=1,
        grid=(NB,),
        in_specs=[
            pl.BlockSpec((BLK, DW), lambda i, s: (i, 0)),
            pl.BlockSpec((1, H, D), lambda i, s: (s[i], 0, 0)),
            pl.BlockSpec((1, 1, H), lambda i, s: (s[i], 0, 0)),
            pl.BlockSpec((1, D, H), lambda i, s: (s[i], 0, 0)),
            pl.BlockSpec((1, 1, D), lambda i, s: (s[i], 0, 0)),
            pl.BlockSpec((1, 1, BLK), lambda i, s: (i, 0, 0)),
        ],
        out_specs=pl.BlockSpec((BLK, D), lambda i, s: (i, 0)),
    )
    return pl.pallas_call(
        _mlp_body,
        grid_spec=grid_spec,
        out_shape=jax.ShapeDtypeStruct((PAD, D), jnp.float32),
    )(blk_exp, x_sorted, W1.astype(jnp.bfloat16), b1.reshape(E, 1, H),
      W2.astype(jnp.bfloat16), b2.reshape(E, 1, D),
      w_pad.reshape(NB, 1, BLK))


_C_RW = N // NW
_C_CH = 16
_C_NCH = _C_RW // _C_CH


def _add_rows(a, b, o):
    def row(rr, cc):
        def grp(j, cc2):
            for u in range(8):
                sl = pl.ds(j * 128 + u * 16, 16)
                o[rr, sl] = a[rr, sl] + b[rr, sl]
            return cc2
        lax.fori_loop(0, D // 128, grp, 0)
        return cc
    lax.fori_loop(0, _C_CH, row, 0)


@functools.lru_cache(maxsize=None)
def _build_sc_combine():
    mesh = plsc.VectorSubcoreMesh(core_axis_name="c", subcore_axis_name="s")

    @functools.partial(
        pl.kernel,
        out_type=jax.ShapeDtypeStruct((N, D), jnp.float32),
        mesh=mesh,
        scratch_types=[
            pltpu.VMEM((2, _C_CH), jnp.int32),
            pltpu.VMEM((2, _C_CH), jnp.int32),
            pltpu.VMEM((_C_CH, D), jnp.float32),
            pltpu.VMEM((_C_CH, D), jnp.float32),
            pltpu.VMEM((_C_CH, D), jnp.float32),
            pltpu.VMEM((_C_CH, D), jnp.float32),
            pltpu.VMEM((_C_CH, D), jnp.float32),
            pltpu.VMEM((_C_CH, D), jnp.float32),
            pltpu.SemaphoreType.DMA,
            pltpu.SemaphoreType.DMA,
            pltpu.SemaphoreType.DMA,
            pltpu.SemaphoreType.DMA,
        ],
    )
    def _sc_combine(y_hbm, p0_hbm, p1_hbm, out_hbm, i0, i1,
                    a0, a1, bb0, bb1, o0, o1, g0, g1, w0, w1):
        wid = lax.axis_index("s") * _SC_NC + lax.axis_index("c")
        base = wid * _C_RW
        abufs = (a0, a1)
        bbufs = (bb0, bb1)
        obufs = (o0, o1)
        gsems = (g0, g1)
        wsems = (w0, w1)
        gops = [None, None]
        wops = [None, None]

        def do_chunk_compute(pb, poff):
            for op in gops[pb]:
                op.wait()
            _add_rows(abufs[pb], bbufs[pb], obufs[pb])
            wops[pb] = tuple(
                pltpu.async_copy(obufs[pb].at[pl.ds(s0, 8)],
                                 out_hbm.at[pl.ds(poff + s0, 8)], wsems[pb])
                for s0 in range(0, _C_CH, 8))

        for c in range(_C_NCH):
            b = c & 1
            off = base + c * _C_CH
            if c >= 2:
                for op in wops[b]:
                    op.wait()
            pltpu.sync_copy(p0_hbm.at[pl.ds(off, _C_CH)], i0.at[b])
            pltpu.sync_copy(p1_hbm.at[pl.ds(off, _C_CH)], i1.at[b])
            ops = []
            for s0 in range(0, _C_CH, 8):
                ops.append(pltpu.async_copy(
                    y_hbm.at[i0.at[b, pl.ds(s0, 8)]],
                    abufs[b].at[pl.ds(s0, 8)], gsems[b]))
                ops.append(pltpu.async_copy(
                    y_hbm.at[i1.at[b, pl.ds(s0, 8)]],
                    bbufs[b].at[pl.ds(s0, 8)], gsems[b]))
            gops[b] = tuple(ops)
            if c >= 1:
                do_chunk_compute(1 - b, base + (c - 1) * _C_CH)
        b = (_C_NCH - 1) & 1
        do_chunk_compute(b, base + (_C_NCH - 1) * _C_CH)
        for op in wops[1 - b]:
            op.wait()
        for op in wops[b]:
            op.wait()

    return _sc_combine


def kernel(x, Wg, bg, W1, b1, W2, b2):
    x_flat = x.reshape(N, D)
    g, e, xw = _gating(x_flat, Wg, bg)
    tok_pad, w_pad, pos, blk_exp = _route(g, e)
    x_sorted = _build_sc_gather()(xw, tok_pad)
    y = _grouped_mlp(x_sorted, W1, b1, W2, b2, w_pad, blk_exp)
    pos0 = pos[0::TOP_K]
    pos1 = pos[1::TOP_K]
    out = _build_sc_combine()(y, pos0, pos1)
    return out.reshape(B, S, D)


# speedup vs baseline: 1.2071x; 1.2071x over previous
import functools

import jax
import jax.numpy as jnp
from jax import lax
from jax.experimental import pallas as pl
from jax.experimental.pallas import tpu as pltpu
from jax.experimental.pallas import tpu_sc as plsc

B, S, D, H, E, TOP_K = 2, 2048, 1024, 2048, 8, 2
N = B * S
A = N * TOP_K
BLK = 128
PAD = A + E * BLK
NB = PAD // BLK
TB = 512
NTB = N // TB

_SC_NC = 2
_SC_NS = 16
NW = _SC_NC * _SC_NS


DW = D // 2


def _pack_round_bf16(b):
    lsb = (b >> 16) & jnp.uint32(1)
    return (b + jnp.uint32(0x7FFF) + lsb) & jnp.uint32(0xFFFF0000)


def _gating_body(x_ref, wg_ref, bg_ref, g_ref, e_ref, xw_ref):
    x = x_ref[...]
    bits = lax.bitcast_convert_type(x, jnp.uint32)
    xw_ref[...] = lax.bitcast_convert_type(
        _pack_round_bf16(bits[:, :DW]) | (_pack_round_bf16(bits[:, DW:]) >> 16),
        jnp.int32)
    logits = lax.dot_general(x, wg_ref[...], (((1,), (1,)), ((), ())),
                             preferred_element_type=jnp.float32)
    logits = logits + bg_ref[...]
    m = jnp.max(logits, axis=1, keepdims=True)
    p = jnp.exp(logits - m)
    probs = p / jnp.sum(p, axis=1, keepdims=True)
    idx = lax.broadcasted_iota(jnp.int32, probs.shape, 1)
    g1 = jnp.max(probs, axis=1)
    a1 = jnp.min(jnp.where(probs == g1[:, None], idx, E), axis=1)
    masked = jnp.where(idx == a1[:, None], -jnp.inf, probs)
    g2 = jnp.max(masked, axis=1)
    a2 = jnp.min(jnp.where(masked == g2[:, None], idx, E), axis=1)
    g_ref[0] = jnp.stack([g1, g2])
    e_ref[0] = jnp.stack([a1, a2])


def _gating(x_flat, Wg, bg):
    g, e, xw = pl.pallas_call(
        _gating_body,
        grid=(NTB,),
        in_specs=[
            pl.BlockSpec((TB, D), lambda i: (i, 0)),
            pl.BlockSpec((E, D), lambda i: (0, 0)),
            pl.BlockSpec((1, E), lambda i: (0, 0)),
        ],
        out_specs=[
            pl.BlockSpec((1, TOP_K, TB), lambda i: (i, 0, 0)),
            pl.BlockSpec((1, TOP_K, TB), lambda i: (i, 0, 0)),
            pl.BlockSpec((TB, DW), lambda i: (i, 0)),
        ],
        out_shape=[
            jax.ShapeDtypeStruct((NTB, TOP_K, TB), jnp.float32),
            jax.ShapeDtypeStruct((NTB, TOP_K, TB), jnp.int32),
            jax.ShapeDtypeStruct((N, DW), jnp.int32),
        ],
    )(x_flat, Wg, bg.reshape(1, E))
    g = jnp.transpose(g, (0, 2, 1)).reshape(N, TOP_K)
    e = jnp.transpose(e, (0, 2, 1)).reshape(N, TOP_K)
    return g, e, xw


def _route(g, e):
    ea = e.reshape(A)
    one_hot = (ea[:, None] == jnp.arange(E, dtype=jnp.int32)[None, :])
    c = jnp.cumsum(one_hot.astype(jnp.int32), axis=0)
    rank = jnp.take_along_axis(c, ea[:, None], axis=1)[:, 0] - 1
    counts = c[-1]
    padded = ((counts + BLK - 1) // BLK) * BLK
    pstarts = jnp.concatenate([jnp.zeros((1,), jnp.int32),
                               jnp.cumsum(padded)[:-1].astype(jnp.int32)])
    pad_pos = pstarts[ea] + rank
    tok_pad = jnp.zeros((PAD,), jnp.int32).at[pad_pos].set(
        (jnp.arange(A, dtype=jnp.int32) // TOP_K))
    w_pad = jnp.zeros((PAD,), jnp.float32).at[pad_pos].set(g.reshape(A))
    pos = pad_pos
    cumb = jnp.cumsum(padded // BLK).astype(jnp.int32)
    blk_exp = jnp.minimum(
        jnp.searchsorted(cumb, jnp.arange(NB, dtype=jnp.int32), side="right"),
        E - 1).astype(jnp.int32)
    return tok_pad, w_pad, pos, blk_exp


_G_RW = PAD // NW
_G_CH = _G_RW // 4
_G_NCH = _G_RW // _G_CH


@functools.lru_cache(maxsize=None)
def _build_sc_gather():
    mesh = plsc.VectorSubcoreMesh(core_axis_name="c", subcore_axis_name="s")

    @functools.partial(
        pl.kernel,
        out_type=jax.ShapeDtypeStruct((PAD, DW), jnp.int32),
        mesh=mesh,
        scratch_types=[
            pltpu.VMEM((2, _G_CH), jnp.int32),
            pltpu.VMEM((_G_CH, DW), jnp.int32),
            pltpu.VMEM((_G_CH, DW), jnp.int32),
            pltpu.SemaphoreType.DMA,
            pltpu.SemaphoreType.DMA,
            pltpu.SemaphoreType.DMA,
            pltpu.SemaphoreType.DMA,
        ],
    )
    def _sc_gather_rows(x_hbm, idx_hbm, out_hbm, idx_v, r0, r1, g0, g1, w0, w1):
        wid = lax.axis_index("s") * _SC_NC + lax.axis_index("c")
        base = wid * _G_RW
        rows = (r0, r1)
        gsems = (g0, g1)
        wsems = (w0, w1)
        gops = [None, None]
        wops = [None, None]

        def start_gather(b):
            ops = []
            for s0 in range(0, _G_CH, 8):
                ops.append(pltpu.async_copy(
                    x_hbm.at[idx_v.at[b, pl.ds(s0, 8)]],
                    rows[b].at[pl.ds(s0, 8)], gsems[b]))
            return ops

        def start_writeback(b, poff):
            ops = []
            for s0 in range(0, _G_CH, 8):
                ops.append(pltpu.async_copy(
                    rows[b].at[pl.ds(s0, 8)],
                    out_hbm.at[pl.ds(poff + s0, 8)], wsems[b]))
            return ops

        for c in range(_G_NCH):
            b = c & 1
            off = base + c * _G_CH
            if c >= 2:
                for op in wops[b]:
                    op.wait()
            pltpu.sync_copy(idx_hbm.at[pl.ds(off, _G_CH)], idx_v.at[b])
            gops[b] = start_gather(b)
            if c >= 1:
                pb = 1 - b
                for op in gops[pb]:
                    op.wait()
                wops[pb] = start_writeback(pb, base + (c - 1) * _G_CH)
        b = (_G_NCH - 1) & 1
        for op in gops[b]:
            op.wait()
        wops[b] = start_writeback(b, base + (_G_NCH - 1) * _G_CH)
        for op in wops[1 - b]:
            op.wait()
        for op in wops[b]:
            op.wait()

    return _sc_gather_rows


def _mlp_body(s_ref, x_ref, w1_ref, b1_ref, w2_ref, b2_ref, w_ref, y_ref):
    u = lax.bitcast_convert_type(x_ref[...], jnp.uint32)
    xa = lax.bitcast_convert_type(u & jnp.uint32(0xFFFF0000), jnp.float32)
    xb = lax.bitcast_convert_type(u << 16, jnp.float32)
    w1 = w1_ref[0]
    h = lax.dot_general(xa, w1[:, :DW], (((1,), (1,)), ((), ())),
                        preferred_element_type=jnp.float32)
    h = h + lax.dot_general(xb, w1[:, DW:], (((1,), (1,)), ((), ())),
                            preferred_element_type=jnp.float32)
    h = jnp.maximum(h + b1_ref[0], 0.0)
    y = lax.dot_general(h, w2_ref[0], (((1,), (1,)), ((), ())),
                        preferred_element_type=jnp.float32)
    y_ref[...] = (y + b2_ref[0]) * w_ref[0, 0][:, None]


def _grouped_mlp(x_sorted, W1, b1, W2, b2, w_pad, blk_exp):
    grid_spec = pltpu.PrefetchScalarGridSpec(
        num_scalar_prefetch=1,
        grid=(NB,),
        in_specs=[
            pl.BlockSpec((BLK, DW), lambda i, s: (i, 0)),
            pl.BlockSpec((1, H, D), lambda i, s: (s[i], 0, 0)),
            pl.BlockSpec((1, 1, H), lambda i, s: (s[i], 0, 0)),
            pl.BlockSpec((1, D, H), lambda i, s: (s[i], 0, 0)),
            pl.BlockSpec((1, 1, D), lambda i, s: (s[i], 0, 0)),
            pl.BlockSpec((1, 1, BLK), lambda i, s: (i, 0, 0)),
        ],
        out_specs=pl.BlockSpec((BLK, D), lambda i, s: (i, 0)),
    )
    return pl.pallas_call(
        _mlp_body,
        grid_spec=grid_spec,
        out_shape=jax.ShapeDtypeStruct((PAD, D), jnp.float32),
    )(blk_exp, x_sorted, W1, b1.reshape(E, 1, H), W2, b2.reshape(E, 1, D),
      w_pad.reshape(NB, 1, BLK))


_C_RW = N // NW
_C_CH = 16
_C_NCH = _C_RW // _C_CH


def _add_rows(a, b, o):
    def row(rr, cc):
        def grp(j, cc2):
            for u in range(8):
                sl = pl.ds(j * 128 + u * 16, 16)
                o[rr, sl] = a[rr, sl] + b[rr, sl]
            return cc2
        lax.fori_loop(0, D // 128, grp, 0)
        return cc
    lax.fori_loop(0, _C_CH, row, 0)


@functools.lru_cache(maxsize=None)
def _build_sc_combine():
    mesh = plsc.VectorSubcoreMesh(core_axis_name="c", subcore_axis_name="s")

    @functools.partial(
        pl.kernel,
        out_type=jax.ShapeDtypeStruct((N, D), jnp.float32),
        mesh=mesh,
        scratch_types=[
            pltpu.VMEM((2, _C_CH), jnp.int32),
            pltpu.VMEM((2, _C_CH), jnp.int32),
            pltpu.VMEM((_C_CH, D), jnp.float32),
            pltpu.VMEM((_C_CH, D), jnp.float32),
            pltpu.VMEM((_C_CH, D), jnp.float32),
            pltpu.VMEM((_C_CH, D), jnp.float32),
            pltpu.VMEM((_C_CH, D), jnp.float32),
            pltpu.VMEM((_C_CH, D), jnp.float32),
            pltpu.SemaphoreType.DMA,
            pltpu.SemaphoreType.DMA,
            pltpu.SemaphoreType.DMA,
            pltpu.SemaphoreType.DMA,
        ],
    )
    def _sc_combine(y_hbm, p0_hbm, p1_hbm, out_hbm, i0, i1,
                    a0, a1, bb0, bb1, o0, o1, g0, g1, w0, w1):
        wid = lax.axis_index("s") * _SC_NC + lax.axis_index("c")
        base = wid * _C_RW
        abufs = (a0, a1)
        bbufs = (bb0, bb1)
        obufs = (o0, o1)
        gsems = (g0, g1)
        wsems = (w0, w1)
        gops = [None, None]
        wops = [None, None]

        def do_chunk_compute(pb, poff):
            for op in gops[pb]:
                op.wait()
            _add_rows(abufs[pb], bbufs[pb], obufs[pb])
            wops[pb] = tuple(
                pltpu.async_copy(obufs[pb].at[pl.ds(s0, 8)],
                                 out_hbm.at[pl.ds(poff + s0, 8)], wsems[pb])
                for s0 in range(0, _C_CH, 8))

        for c in range(_C_NCH):
            b = c & 1
            off = base + c * _C_CH
            if c >= 2:
                for op in wops[b]:
                    op.wait()
            pltpu.sync_copy(p0_hbm.at[pl.ds(off, _C_CH)], i0.at[b])
            pltpu.sync_copy(p1_hbm.at[pl.ds(off, _C_CH)], i1.at[b])
            ops = []
            for s0 in range(0, _C_CH, 8):
                ops.append(pltpu.async_copy(
                    y_hbm.at[i0.at[b, pl.ds(s0, 8)]],
                    abufs[b].at[pl.ds(s0, 8)], gsems[b]))
                ops.append(pltpu.async_copy(
                    y_hbm.at[i1.at[b, pl.ds(s0, 8)]],
                    bbufs[b].at[pl.ds(s0, 8)], gsems[b]))
            gops[b] = tuple(ops)
            if c >= 1:
                do_chunk_compute(1 - b, base + (c - 1) * _C_CH)
        b = (_C_NCH - 1) & 1
        do_chunk_compute(b, base + (_C_NCH - 1) * _C_CH)
        for op in wops[1 - b]:
            op.wait()
        for op in wops[b]:
            op.wait()

    return _sc_combine


def kernel(x, Wg, bg, W1, b1, W2, b2):
    x_flat = x.reshape(N, D)
    g, e, xw = _gating(x_flat, Wg, bg)
    tok_pad, w_pad, pos, blk_exp = _route(g, e)
    x_sorted = _build_sc_gather()(xw, tok_pad)
    y = _grouped_mlp(x_sorted, W1, b1, W2, b2, w_pad, blk_exp)
    pos0 = pos[0::TOP_K]
    pos1 = pos[1::TOP_K]
    out = _build_sc_combine()(y, pos0, pos1)
    return out.reshape(B, S, D)


# speedup vs baseline: 1.4144x; 1.1717x over previous
import functools

import jax
import jax.numpy as jnp
from jax import lax
from jax.experimental import pallas as pl
from jax.experimental.pallas import tpu as pltpu
from jax.experimental.pallas import tpu_sc as plsc

B, S, D, H, E, TOP_K = 2, 2048, 1024, 2048, 8, 2
N = B * S
A = N * TOP_K
BLK = 256
PAD = A + E * BLK
NB = PAD // BLK
TB = 512
NTB = N // TB

_SC_NC = 2
_SC_NS = 16
NW = _SC_NC * _SC_NS


DW = D // 2


def _pack_round_bf16(b):
    lsb = (b >> 16) & jnp.uint32(1)
    return (b + jnp.uint32(0x7FFF) + lsb) & jnp.uint32(0xFFFF0000)


def _gating_body(x_ref, wg_ref, bg_ref, g_ref, e_ref, xw_ref):
    x = x_ref[...]
    bits = lax.bitcast_convert_type(x, jnp.uint32)
    xw_ref[...] = lax.bitcast_convert_type(
        _pack_round_bf16(bits[:, :DW]) | (_pack_round_bf16(bits[:, DW:]) >> 16),
        jnp.int32)
    logits = lax.dot_general(x, wg_ref[...], (((1,), (1,)), ((), ())),
                             preferred_element_type=jnp.float32)
    logits = logits + bg_ref[...]
    m = jnp.max(logits, axis=1, keepdims=True)
    p = jnp.exp(logits - m)
    probs = p / jnp.sum(p, axis=1, keepdims=True)
    idx = lax.broadcasted_iota(jnp.int32, probs.shape, 1)
    g1 = jnp.max(probs, axis=1)
    a1 = jnp.min(jnp.where(probs == g1[:, None], idx, E), axis=1)
    masked = jnp.where(idx == a1[:, None], -jnp.inf, probs)
    g2 = jnp.max(masked, axis=1)
    a2 = jnp.min(jnp.where(masked == g2[:, None], idx, E), axis=1)
    g_ref[0] = jnp.stack([g1, g2])
    e_ref[0] = jnp.stack([a1, a2])


def _gating(x_flat, Wg, bg):
    g, e, xw = pl.pallas_call(
        _gating_body,
        grid=(NTB,),
        in_specs=[
            pl.BlockSpec((TB, D), lambda i: (i, 0)),
            pl.BlockSpec((E, D), lambda i: (0, 0)),
            pl.BlockSpec((1, E), lambda i: (0, 0)),
        ],
        out_specs=[
            pl.BlockSpec((1, TOP_K, TB), lambda i: (i, 0, 0)),
            pl.BlockSpec((1, TOP_K, TB), lambda i: (i, 0, 0)),
            pl.BlockSpec((TB, DW), lambda i: (i, 0)),
        ],
        out_shape=[
            jax.ShapeDtypeStruct((NTB, TOP_K, TB), jnp.float32),
            jax.ShapeDtypeStruct((NTB, TOP_K, TB), jnp.int32),
            jax.ShapeDtypeStruct((N, DW), jnp.int32),
        ],
    )(x_flat, Wg, bg.reshape(1, E))
    g = jnp.transpose(g, (0, 2, 1)).reshape(N, TOP_K)
    e = jnp.transpose(e, (0, 2, 1)).reshape(N, TOP_K)
    return g, e, xw


def _route(g, e):
    ea = e.reshape(A)
    one_hot = (ea[:, None] == jnp.arange(E, dtype=jnp.int32)[None, :])
    c = jnp.cumsum(one_hot.astype(jnp.int32), axis=0)
    rank = jnp.take_along_axis(c, ea[:, None], axis=1)[:, 0] - 1
    counts = c[-1]
    padded = ((counts + BLK - 1) // BLK) * BLK
    pstarts = jnp.concatenate([jnp.zeros((1,), jnp.int32),
                               jnp.cumsum(padded)[:-1].astype(jnp.int32)])
    pad_pos = pstarts[ea] + rank
    tok_pad = jnp.zeros((PAD,), jnp.int32).at[pad_pos].set(
        (jnp.arange(A, dtype=jnp.int32) // TOP_K))
    w_pad = jnp.zeros((PAD,), jnp.float32).at[pad_pos].set(g.reshape(A))
    pos = pad_pos
    cumb = jnp.cumsum(padded // BLK).astype(jnp.int32)
    blk_exp = jnp.minimum(
        jnp.searchsorted(cumb, jnp.arange(NB, dtype=jnp.int32), side="right"),
        E - 1).astype(jnp.int32)
    return tok_pad, w_pad, pos, blk_exp


_G_RW = PAD // NW
_G_CH = _G_RW // 4
_G_NCH = _G_RW // _G_CH


@functools.lru_cache(maxsize=None)
def _build_sc_gather():
    mesh = plsc.VectorSubcoreMesh(core_axis_name="c", subcore_axis_name="s")

    @functools.partial(
        pl.kernel,
        out_type=jax.ShapeDtypeStruct((PAD, DW), jnp.int32),
        mesh=mesh,
        scratch_types=[
            pltpu.VMEM((2, _G_CH), jnp.int32),
            pltpu.VMEM((_G_CH, DW), jnp.int32),
            pltpu.VMEM((_G_CH, DW), jnp.int32),
            pltpu.SemaphoreType.DMA,
            pltpu.SemaphoreType.DMA,
            pltpu.SemaphoreType.DMA,
            pltpu.SemaphoreType.DMA,
        ],
    )
    def _sc_gather_rows(x_hbm, idx_hbm, out_hbm, idx_v, r0, r1, g0, g1, w0, w1):
        wid = lax.axis_index("s") * _SC_NC + lax.axis_index("c")
        base = wid * _G_RW
        rows = (r0, r1)
        gsems = (g0, g1)
        wsems = (w0, w1)
        gops = [None, None]
        wops = [None, None]

        def start_gather(b):
            ops = []
            for s0 in range(0, _G_CH, 8):
                ops.append(pltpu.async_copy(
                    x_hbm.at[idx_v.at[b, pl.ds(s0, 8)]],
                    rows[b].at[pl.ds(s0, 8)], gsems[b]))
            return ops

        def start_writeback(b, poff):
            ops = []
            for s0 in range(0, _G_CH, 8):
                ops.append(pltpu.async_copy(
                    rows[b].at[pl.ds(s0, 8)],
                    out_hbm.at[pl.ds(poff + s0, 8)], wsems[b]))
            return ops

        for c in range(_G_NCH):
            b = c & 1
            off = base + c * _G_CH
            if c >= 2:
                for op in wops[b]:
                    op.wait()
            pltpu.sync_copy(idx_hbm.at[pl.ds(off, _G_CH)], idx_v.at[b])
            gops[b] = start_gather(b)
            if c >= 1:
                pb = 1 - b
                for op in gops[pb]:
                    op.wait()
                wops[pb] = start_writeback(pb, base + (c - 1) * _G_CH)
        b = (_G_NCH - 1) & 1
        for op in gops[b]:
            op.wait()
        wops[b] = start_writeback(b, base + (_G_NCH - 1) * _G_CH)
        for op in wops[1 - b]:
            op.wait()
        for op in wops[b]:
            op.wait()

    return _sc_gather_rows


def _mlp_body(s_ref, x_ref, w1_ref, b1_ref, w2_ref, b2_ref, w_ref, y_ref):
    u = lax.bitcast_convert_type(x_ref[...], jnp.uint32)
    xa = lax.bitcast_convert_type(u & jnp.uint32(0xFFFF0000), jnp.float32)
    xb = lax.bitcast_convert_type(u << 16, jnp.float32)
    w1 = w1_ref[0]
    h = lax.dot_general(xa, w1[:, :DW], (((1,), (1,)), ((), ())),
                        preferred_element_type=jnp.float32)
    h = h + lax.dot_general(xb, w1[:, DW:], (((1,), (1,)), ((), ())),
                            preferred_element_type=jnp.float32)
    h = jnp.maximum(h + b1_ref[0], 0.0)
    y = lax.dot_general(h, w2_ref[0], (((1,), (1,)), ((), ())),
                        preferred_element_type=jnp.float32)
    y_ref[...] = (y + b2_ref[0]) * w_ref[0, 0][:, None]


def _grouped_mlp(x_sorted, W1, b1, W2, b2, w_pad, blk_exp):
    grid_spec = pltpu.PrefetchScalarGridSpec(
        num_scalar_prefetch=1,
        grid=(NB,),
        in_specs=[
            pl.BlockSpec((BLK, DW), lambda i, s: (i, 0)),
            pl.BlockSpec((1, H, D), lambda i, s: (s[i], 0, 0)),
            pl.BlockSpec((1, 1, H), lambda i, s: (s[i], 0, 0)),
            pl.BlockSpec((1, D, H), lambda i, s: (s[i], 0, 0)),
            pl.BlockSpec((1, 1, D), lambda i, s: (s[i], 0, 0)),
            pl.BlockSpec((1, 1, BLK), lambda i, s: (i, 0, 0)),
        ],
        out_specs=pl.BlockSpec((BLK, D), lambda i, s: (i, 0)),
    )
    return pl.pallas_call(
        _mlp_body,
        grid_spec=grid_spec,
        out_shape=jax.ShapeDtypeStruct((PAD, D), jnp.float32),
    )(blk_exp, x_sorted, W1, b1.reshape(E, 1, H), W2, b2.reshape(E, 1, D),
      w_pad.reshape(NB, 1, BLK))


_C_RW = N // NW
_C_CH = 16
_C_NCH = _C_RW // _C_CH


def _add_rows(a, b, o):
    def row(rr, cc):
        def grp(j, cc2):
            for u in range(8):
                sl = pl.ds(j * 128 + u * 16, 16)
                o[rr, sl] = a[rr, sl] + b[rr, sl]
            return cc2
        lax.fori_loop(0, D // 128, grp, 0)
        return cc
    lax.fori_loop(0, _C_CH, row, 0)


@functools.lru_cache(maxsize=None)
def _build_sc_combine():
    mesh = plsc.VectorSubcoreMesh(core_axis_name="c", subcore_axis_name="s")

    @functools.partial(
        pl.kernel,
        out_type=jax.ShapeDtypeStruct((N, D), jnp.float32),
        mesh=mesh,
        scratch_types=[
            pltpu.VMEM((2, _C_CH), jnp.int32),
            pltpu.VMEM((2, _C_CH), jnp.int32),
            pltpu.VMEM((_C_CH, D), jnp.float32),
            pltpu.VMEM((_C_CH, D), jnp.float32),
            pltpu.VMEM((_C_CH, D), jnp.float32),
            pltpu.VMEM((_C_CH, D), jnp.float32),
            pltpu.VMEM((_C_CH, D), jnp.float32),
            pltpu.VMEM((_C_CH, D), jnp.float32),
            pltpu.SemaphoreType.DMA,
            pltpu.SemaphoreType.DMA,
            pltpu.SemaphoreType.DMA,
            pltpu.SemaphoreType.DMA,
        ],
    )
    def _sc_combine(y_hbm, p0_hbm, p1_hbm, out_hbm, i0, i1,
                    a0, a1, bb0, bb1, o0, o1, g0, g1, w0, w1):
        wid = lax.axis_index("s") * _SC_NC + lax.axis_index("c")
        base = wid * _C_RW
        abufs = (a0, a1)
        bbufs = (bb0, bb1)
        obufs = (o0, o1)
        gsems = (g0, g1)
        wsems = (w0, w1)
        gops = [None, None]
        wops = [None, None]

        def do_chunk_compute(pb, poff):
            for op in gops[pb]:
                op.wait()
            _add_rows(abufs[pb], bbufs[pb], obufs[pb])
            wops[pb] = tuple(
                pltpu.async_copy(obufs[pb].at[pl.ds(s0, 8)],
                                 out_hbm.at[pl.ds(poff + s0, 8)], wsems[pb])
                for s0 in range(0, _C_CH, 8))

        for c in range(_C_NCH):
            b = c & 1
            off = base + c * _C_CH
            if c >= 2:
                for op in wops[b]:
                    op.wait()
            pltpu.sync_copy(p0_hbm.at[pl.ds(off, _C_CH)], i0.at[b])
            pltpu.sync_copy(p1_hbm.at[pl.ds(off, _C_CH)], i1.at[b])
            ops = []
            for s0 in range(0, _C_CH, 8):
                ops.append(pltpu.async_copy(
                    y_hbm.at[i0.at[b, pl.ds(s0, 8)]],
                    abufs[b].at[pl.ds(s0, 8)], gsems[b]))
                ops.append(pltpu.async_copy(
                    y_hbm.at[i1.at[b, pl.ds(s0, 8)]],
                    bbufs[b].at[pl.ds(s0, 8)], gsems[b]))
            gops[b] = tuple(ops)
            if c >= 1:
                do_chunk_compute(1 - b, base + (c - 1) * _C_CH)
        b = (_C_NCH - 1) & 1
        do_chunk_compute(b, base + (_C_NCH - 1) * _C_CH)
        for op in wops[1 - b]:
            op.wait()
        for op in wops[b]:
            op.wait()

    return _sc_combine


def kernel(x, Wg, bg, W1, b1, W2, b2):
    x_flat = x.reshape(N, D)
    g, e, xw = _gating(x_flat, Wg, bg)
    tok_pad, w_pad, pos, blk_exp = _route(g, e)
    x_sorted = _build_sc_gather()(xw, tok_pad)
    y = _grouped_mlp(x_sorted, W1, b1, W2, b2, w_pad, blk_exp)
    pos0 = pos[0::TOP_K]
    pos1 = pos[1::TOP_K]
    out = _build_sc_combine()(y, pos0, pos1)
    return out.reshape(B, S, D)


# speedup vs baseline: 1.8007x; 1.2731x over previous
import functools

import jax
import jax.numpy as jnp
from jax import lax
from jax.experimental import pallas as pl
from jax.experimental.pallas import tpu as pltpu
from jax.experimental.pallas import tpu_sc as plsc

B, S, D, H, E, TOP_K = 2, 2048, 1024, 2048, 8, 2
N = B * S
A = N * TOP_K
BLK = 256
PAD = A + E * BLK
NB = PAD // BLK
TB = 512
NTB = N // TB

_SC_NC = 2
_SC_NS = 16
NW = _SC_NC * _SC_NS


DW = D // 2


def _pack_round_bf16(b):
    lsb = (b >> 16) & jnp.uint32(1)
    return (b + jnp.uint32(0x7FFF) + lsb) & jnp.uint32(0xFFFF0000)


def _gating_body(x_ref, wg_ref, bg_ref, g_ref, e_ref, xw_ref):
    x = x_ref[...]
    bits = lax.bitcast_convert_type(x, jnp.uint32)
    xw_ref[...] = lax.bitcast_convert_type(
        _pack_round_bf16(bits[:, :DW]) | (_pack_round_bf16(bits[:, DW:]) >> 16),
        jnp.int32)
    logits = lax.dot_general(x, wg_ref[...], (((1,), (1,)), ((), ())),
                             preferred_element_type=jnp.float32)
    logits = logits + bg_ref[...]
    m = jnp.max(logits, axis=1, keepdims=True)
    p = jnp.exp(logits - m)
    probs = p / jnp.sum(p, axis=1, keepdims=True)
    idx = lax.broadcasted_iota(jnp.int32, probs.shape, 1)
    g1 = jnp.max(probs, axis=1)
    a1 = jnp.min(jnp.where(probs == g1[:, None], idx, E), axis=1)
    masked = jnp.where(idx == a1[:, None], -jnp.inf, probs)
    g2 = jnp.max(masked, axis=1)
    a2 = jnp.min(jnp.where(masked == g2[:, None], idx, E), axis=1)
    g_ref[0] = jnp.stack([g1, g2])
    e_ref[0] = jnp.stack([a1, a2])


def _gating(x_flat, Wg, bg):
    g, e, xw = pl.pallas_call(
        _gating_body,
        grid=(NTB,),
        in_specs=[
            pl.BlockSpec((TB, D), lambda i: (i, 0)),
            pl.BlockSpec((E, D), lambda i: (0, 0)),
            pl.BlockSpec((1, E), lambda i: (0, 0)),
        ],
        out_specs=[
            pl.BlockSpec((1, TOP_K, TB), lambda i: (i, 0, 0)),
            pl.BlockSpec((1, TOP_K, TB), lambda i: (i, 0, 0)),
            pl.BlockSpec((TB, DW), lambda i: (i, 0)),
        ],
        out_shape=[
            jax.ShapeDtypeStruct((NTB, TOP_K, TB), jnp.float32),
            jax.ShapeDtypeStruct((NTB, TOP_K, TB), jnp.int32),
            jax.ShapeDtypeStruct((N, DW), jnp.int32),
        ],
    )(x_flat, Wg, bg.reshape(1, E))
    g = jnp.transpose(g, (0, 2, 1)).reshape(N, TOP_K)
    e = jnp.transpose(e, (0, 2, 1)).reshape(N, TOP_K)
    return g, e, xw


def _route(g, e):
    ea = e.reshape(A)
    one_hot = (ea[:, None] == jnp.arange(E, dtype=jnp.int32)[None, :])
    c = jnp.cumsum(one_hot.astype(jnp.int32), axis=0)
    rank = jnp.take_along_axis(c, ea[:, None], axis=1)[:, 0] - 1
    counts = c[-1]
    padded = ((counts + BLK - 1) // BLK) * BLK
    pstarts = jnp.concatenate([jnp.zeros((1,), jnp.int32),
                               jnp.cumsum(padded)[:-1].astype(jnp.int32)])
    pad_pos = pstarts[ea] + rank
    tok_pad = jnp.zeros((PAD,), jnp.int32).at[pad_pos].set(
        (jnp.arange(A, dtype=jnp.int32) // TOP_K))
    w_pad = jnp.zeros((PAD,), jnp.float32).at[pad_pos].set(g.reshape(A))
    pos = pad_pos
    cumb = jnp.cumsum(padded // BLK).astype(jnp.int32)
    blk_exp = jnp.minimum(
        jnp.searchsorted(cumb, jnp.arange(NB, dtype=jnp.int32), side="right"),
        E - 1).astype(jnp.int32)
    return tok_pad, w_pad, pos, blk_exp


_G_RW = N // NW
_G_CH = 16
_G_NCH = _G_RW // _G_CH


@functools.lru_cache(maxsize=None)
def _build_sc_scatter():
    mesh = plsc.VectorSubcoreMesh(core_axis_name="c", subcore_axis_name="s")

    @functools.partial(
        pl.kernel,
        out_type=jax.ShapeDtypeStruct((PAD, DW), jnp.int32),
        mesh=mesh,
        scratch_types=[
            pltpu.VMEM((2, _G_CH), jnp.int32),
            pltpu.VMEM((2, _G_CH), jnp.int32),
            pltpu.VMEM((_G_CH, DW), jnp.int32),
            pltpu.VMEM((_G_CH, DW), jnp.int32),
            pltpu.SemaphoreType.DMA,
            pltpu.SemaphoreType.DMA,
        ],
    )
    def _sc_scatter_rows(x_hbm, p0_hbm, p1_hbm, out_hbm,
                         i0, i1, r0, r1, w0, w1):
        wid = lax.axis_index("s") * _SC_NC + lax.axis_index("c")
        base = wid * _G_RW
        rows = (r0, r1)
        idx0 = (i0, i1)
        wsems = (w0, w1)
        wops = [None, None]

        for c in range(_G_NCH):
            b = c & 1
            off = base + c * _G_CH
            if c >= 2:
                for op in wops[b]:
                    op.wait()
            pltpu.sync_copy(p0_hbm.at[pl.ds(off, _G_CH)], idx0[0].at[b])
            pltpu.sync_copy(p1_hbm.at[pl.ds(off, _G_CH)], idx0[1].at[b])
            pltpu.sync_copy(x_hbm.at[pl.ds(off, _G_CH)], rows[b])
            ops = []
            for s0 in range(0, _G_CH, 8):
                ops.append(pltpu.async_copy(
                    rows[b].at[pl.ds(s0, 8)],
                    out_hbm.at[idx0[0].at[b, pl.ds(s0, 8)]], wsems[b]))
                ops.append(pltpu.async_copy(
                    rows[b].at[pl.ds(s0, 8)],
                    out_hbm.at[idx0[1].at[b, pl.ds(s0, 8)]], wsems[b]))
            wops[b] = ops
        for op in wops[(_G_NCH - 1) & 1]:
            op.wait()
        for op in wops[_G_NCH & 1]:
            op.wait()

    return _sc_scatter_rows


def _mlp_body(s_ref, x_ref, w1_ref, b1_ref, w2_ref, b2_ref, w_ref, y_ref):
    u = lax.bitcast_convert_type(x_ref[...], jnp.uint32)
    xa = lax.bitcast_convert_type(u & jnp.uint32(0xFFFF0000), jnp.float32)
    xb = lax.bitcast_convert_type(u << 16, jnp.float32)
    w1 = w1_ref[0]
    h = lax.dot_general(xa, w1[:, :DW], (((1,), (1,)), ((), ())),
                        preferred_element_type=jnp.float32)
    h = h + lax.dot_general(xb, w1[:, DW:], (((1,), (1,)), ((), ())),
                            preferred_element_type=jnp.float32)
    h = jnp.maximum(h + b1_ref[0], 0.0)
    y = lax.dot_general(h, w2_ref[0], (((1,), (1,)), ((), ())),
                        preferred_element_type=jnp.float32)
    y_ref[...] = (y + b2_ref[0]) * w_ref[0, 0][:, None]


def _grouped_mlp(x_sorted, W1, b1, W2, b2, w_pad, blk_exp):
    grid_spec = pltpu.PrefetchScalarGridSpec(
        num_scalar_prefetch=1,
        grid=(NB,),
        in_specs=[
            pl.BlockSpec((BLK, DW), lambda i, s: (i, 0)),
            pl.BlockSpec((1, H, D), lambda i, s: (s[i], 0, 0)),
            pl.BlockSpec((1, 1, H), lambda i, s: (s[i], 0, 0)),
            pl.BlockSpec((1, D, H), lambda i, s: (s[i], 0, 0)),
            pl.BlockSpec((1, 1, D), lambda i, s: (s[i], 0, 0)),
            pl.BlockSpec((1, 1, BLK), lambda i, s: (i, 0, 0)),
        ],
        out_specs=pl.BlockSpec((BLK, D), lambda i, s: (i, 0)),
    )
    return pl.pallas_call(
        _mlp_body,
        grid_spec=grid_spec,
        out_shape=jax.ShapeDtypeStruct((PAD, D), jnp.float32),
    )(blk_exp, x_sorted, W1, b1.reshape(E, 1, H), W2, b2.reshape(E, 1, D),
      w_pad.reshape(NB, 1, BLK))


_C_RW = N // NW
_C_CH = 16
_C_NCH = _C_RW // _C_CH


def _add_rows(a, b, o):
    def row(rr, cc):
        def grp(j, cc2):
            for u in range(8):
                sl = pl.ds(j * 128 + u * 16, 16)
                o[rr, sl] = a[rr, sl] + b[rr, sl]
            return cc2
        lax.fori_loop(0, D // 128, grp, 0)
        return cc
    lax.fori_loop(0, _C_CH, row, 0)


@functools.lru_cache(maxsize=None)
def _build_sc_combine():
    mesh = plsc.VectorSubcoreMesh(core_axis_name="c", subcore_axis_name="s")

    @functools.partial(
        pl.kernel,
        out_type=jax.ShapeDtypeStruct((N, D), jnp.float32),
        mesh=mesh,
        scratch_types=[
            pltpu.VMEM((2, _C_CH), jnp.int32),
            pltpu.VMEM((2, _C_CH), jnp.int32),
            pltpu.VMEM((_C_CH, D), jnp.float32),
            pltpu.VMEM((_C_CH, D), jnp.float32),
            pltpu.VMEM((_C_CH, D), jnp.float32),
            pltpu.VMEM((_C_CH, D), jnp.float32),
            pltpu.VMEM((_C_CH, D), jnp.float32),
            pltpu.VMEM((_C_CH, D), jnp.float32),
            pltpu.SemaphoreType.DMA,
            pltpu.SemaphoreType.DMA,
            pltpu.SemaphoreType.DMA,
            pltpu.SemaphoreType.DMA,
        ],
    )
    def _sc_combine(y_hbm, p0_hbm, p1_hbm, out_hbm, i0, i1,
                    a0, a1, bb0, bb1, o0, o1, g0, g1, w0, w1):
        wid = lax.axis_index("s") * _SC_NC + lax.axis_index("c")
        base = wid * _C_RW
        abufs = (a0, a1)
        bbufs = (bb0, bb1)
        obufs = (o0, o1)
        gsems = (g0, g1)
        wsems = (w0, w1)
        gops = [None, None]
        wops = [None, None]

        def do_chunk_compute(pb, poff):
            for op in gops[pb]:
                op.wait()
            _add_rows(abufs[pb], bbufs[pb], obufs[pb])
            wops[pb] = tuple(
                pltpu.async_copy(obufs[pb].at[pl.ds(s0, 8)],
                                 out_hbm.at[pl.ds(poff + s0, 8)], wsems[pb])
                for s0 in range(0, _C_CH, 8))

        for c in range(_C_NCH):
            b = c & 1
            off = base + c * _C_CH
            if c >= 2:
                for op in wops[b]:
                    op.wait()
            pltpu.sync_copy(p0_hbm.at[pl.ds(off, _C_CH)], i0.at[b])
            pltpu.sync_copy(p1_hbm.at[pl.ds(off, _C_CH)], i1.at[b])
            ops = []
            for s0 in range(0, _C_CH, 8):
                ops.append(pltpu.async_copy(
                    y_hbm.at[i0.at[b, pl.ds(s0, 8)]],
                    abufs[b].at[pl.ds(s0, 8)], gsems[b]))
                ops.append(pltpu.async_copy(
                    y_hbm.at[i1.at[b, pl.ds(s0, 8)]],
                    bbufs[b].at[pl.ds(s0, 8)], gsems[b]))
            gops[b] = tuple(ops)
            if c >= 1:
                do_chunk_compute(1 - b, base + (c - 1) * _C_CH)
        b = (_C_NCH - 1) & 1
        do_chunk_compute(b, base + (_C_NCH - 1) * _C_CH)
        for op in wops[1 - b]:
            op.wait()
        for op in wops[b]:
            op.wait()

    return _sc_combine


def kernel(x, Wg, bg, W1, b1, W2, b2):
    x_flat = x.reshape(N, D)
    g, e, xw = _gating(x_flat, Wg, bg)
    tok_pad, w_pad, pos, blk_exp = _route(g, e)
    pos0 = pos[0::TOP_K]
    pos1 = pos[1::TOP_K]
    x_sorted = _build_sc_scatter()(xw, pos0, pos1)
    y = _grouped_mlp(x_sorted, W1, b1, W2, b2, w_pad, blk_exp)
    out = _build_sc_combine()(y, pos0, pos1)
    return out.reshape(B, S, D)
